# baseline (speedup 1.0000x reference)
import functools

import jax
import jax.numpy as jnp
from jax import lax
from jax.experimental import pallas as pl
from jax.experimental.pallas import tpu as pltpu
from jax.experimental.pallas import tpu_sc as plsc

N_NODES = 50000
N_EDGES = 800000
HIDDEN = 64
HALF = 32
NUM_GRAPHS = 128
STEPS = 3

NUM_CORES = 2
NUM_SUBCORES = 16

EC = 250
N_CHUNKS = N_EDGES // EC
CHUNKS_PER_TILE = N_CHUNKS // NUM_SUBCORES
IBLK = 4
N_IBLKS = CHUNKS_PER_TILE // IBLK
NBUF = 3

ROWS_PER_TILE = N_NODES // NUM_SUBCORES
WB = 125
WB_PER_TILE = ROWS_PER_TILE // WB

XC = 200
X_CHUNKS = N_NODES // XC
X_ITERS = -(-X_CHUNKS // (NUM_CORES * NUM_SUBCORES))

BLK = 5000
N_BLOCKS = N_NODES // BLK

_sc_mesh = plsc.VectorSubcoreMesh(
    core_axis_name="c", subcore_axis_name="s",
    num_cores=NUM_CORES, num_subcores=NUM_SUBCORES)

_sc_params = pltpu.CompilerParams(use_tc_tiling_on_sc=False)


@functools.partial(
    pl.kernel,
    out_type=(
        jax.ShapeDtypeStruct((N_NODES, HIDDEN), jnp.float32),
        jax.ShapeDtypeStruct((N_NODES, HALF), jnp.float32),
        jax.ShapeDtypeStruct((N_NODES, HALF), jnp.float32),
    ),
    mesh=_sc_mesh,
    compiler_params=_sc_params,
    scratch_types=[
        pltpu.VMEM((XC,), jnp.int32),
        pltpu.VMEM((XC, HIDDEN), jnp.float32),
        pltpu.VMEM((XC, HALF), jnp.float32),
        pltpu.VMEM((XC, HALF), jnp.float32),
        pltpu.SemaphoreType.DMA,
        pltpu.SemaphoreType.DMA,
        pltpu.SemaphoreType.DMA,
    ],
)
def _embed_kernel(x2d_hbm, table_hbm, t0a_hbm, t0b_hbm,
                  h_hbm, ma_hbm, mb_hbm,
                  idx_v, rows_v, ra_v, rb_v, sem, sema, semb):
  c = lax.axis_index("c")
  s = lax.axis_index("s")
  w = s * NUM_CORES + c

  @pl.loop(0, X_ITERS)
  def _(k):
    ci = k * (NUM_CORES * NUM_SUBCORES) + w

    @pl.when(ci < X_CHUNKS)
    def _():
      pltpu.sync_copy(x2d_hbm.at[ci], idx_v)
      ga = pltpu.async_copy(t0a_hbm.at[idx_v], ra_v, sema)
      gb = pltpu.async_copy(t0b_hbm.at[idx_v], rb_v, semb)
      gh = pltpu.async_copy(table_hbm.at[idx_v], rows_v, sem)
      ga.wait()
      pltpu.async_copy(ra_v, ma_hbm.at[pl.ds(ci * XC, XC)], sema).wait()
      gb.wait()
      pltpu.async_copy(rb_v, mb_hbm.at[pl.ds(ci * XC, XC)], semb).wait()
      gh.wait()
      pltpu.async_copy(rows_v, h_hbm.at[pl.ds(ci * XC, XC)], sem).wait()


@functools.partial(
    pl.kernel,
    out_type=(
        jax.ShapeDtypeStruct((N_NODES, HALF), jnp.float32),
        jax.ShapeDtypeStruct((N_NODES, HALF), jnp.float32),
    ),
    mesh=_sc_mesh,
    compiler_params=_sc_params,
    scratch_types=[
        pltpu.VMEM_SHARED((N_NODES, HALF), jnp.float32),
        pltpu.VMEM((NBUF, IBLK, EC), jnp.int32),
        pltpu.VMEM((NBUF, IBLK, EC), jnp.int32),
        pltpu.VMEM((NBUF, EC, HALF), jnp.float32),
        pltpu.SemaphoreType.DMA((NBUF,)),
        pltpu.SemaphoreType.DMA((NBUF,)),
        pltpu.SemaphoreType.DMA((NBUF,)),
    ],
)
def _scatter_kernel(m_a_hbm, m_b_hbm, src_hbm, dst_hbm,
                    agg_a_hbm, agg_b_hbm,
                    acc_spmem, sidx_v, didx_v, rows_v, isem, gsem, ssem):
  c = lax.axis_index("c")
  s = lax.axis_index("s")

  z16 = jnp.zeros((16,), jnp.float32)

  @pl.loop(0, EC)
  def _(i):
    rows_v[0, i, pl.ds(0, 16)] = z16
    rows_v[0, i, pl.ds(16, 16)] = z16

  zrow = rows_v.at[0, pl.ds(0, WB)]

  def _zdesc(q):
    base = s * ROWS_PER_TILE + q * WB
    return pltpu.make_async_copy(zrow, acc_spmem.at[pl.ds(base, WB)],
                                 gsem.at[0])

  @pl.loop(0, WB_PER_TILE)
  def _(q):
    _zdesc(q).start()

  @pl.loop(0, WB_PER_TILE)
  def _(q):
    _zdesc(q).wait()

  plsc.subcore_barrier()

  def run(m_hbm, agg_hbm):
    chunk0 = s * CHUNKS_PER_TILE

    def _idesc(blk, ib):
      c0 = chunk0 + blk * IBLK
      return (
          pltpu.make_async_copy(src_hbm.at[pl.ds(c0, IBLK)], sidx_v.at[ib],
                                isem.at[ib]),
          pltpu.make_async_copy(dst_hbm.at[pl.ds(c0, IBLK)], didx_v.at[ib],
                                isem.at[ib]),
      )

    def _gdesc(b, ib, j):
      return pltpu.make_async_copy(m_hbm.at[sidx_v.at[ib, j]], rows_v.at[b],
                                   gsem.at[b])

    def _sdesc(b, ib, j):
      return pltpu.make_async_copy(rows_v.at[b], acc_spmem.at[didx_v.at[ib, j]],
                                   ssem.at[b])

    for d in _idesc(0, 0):
      d.start()

    @pl.loop(0, N_IBLKS)
    def _(blk):
      ib = lax.rem(blk, NBUF)
      ib_prev = lax.rem(blk + (NBUF - 1), NBUF)
      for d in _idesc(blk, ib):
        d.wait()

      @pl.when(blk + 1 < N_IBLKS)
      def _():
        for d in _idesc(blk + 1, lax.rem(blk + 1, NBUF)):
          d.start()

      for j in range(IBLK):
        k = blk * IBLK + j
        b = lax.rem(k, NBUF)

        if j >= NBUF:
          _sdesc(b, ib, j - NBUF).wait()
        else:
          @pl.when(blk >= 1)
          def _():
            _sdesc(b, ib_prev, j + IBLK - NBUF).wait()

        _gdesc(b, ib, j).start()

        if j >= 2:
          b2 = lax.rem(k - 2, NBUF)
          _gdesc(b2, ib, j - 2).wait()
          _sdesc(b2, ib, j - 2).start(add=True)
        else:
          @pl.when(blk >= 1)
          def _():
            b2 = lax.rem(k + (NBUF - 2), NBUF)
            _gdesc(b2, ib_prev, j + IBLK - 2).wait()
            _sdesc(b2, ib_prev, j + IBLK - 2).start(add=True)

    ib_last = (N_IBLKS - 1) % NBUF
    for k in (CHUNKS_PER_TILE - 2, CHUNKS_PER_TILE - 1):
      b = k % NBUF
      j = k - (N_IBLKS - 1) * IBLK
      _gdesc(b, ib_last, j).wait()
      _sdesc(b, ib_last, j).start(add=True)
    for k in range(CHUNKS_PER_TILE - NBUF, CHUNKS_PER_TILE):
      b = k % NBUF
      j = k - (N_IBLKS - 1) * IBLK
      _sdesc(b, ib_last, max(j, 0)).wait()

    plsc.subcore_barrier()

    def _rdesc(q, b):
      base = s * ROWS_PER_TILE + q * WB
      return pltpu.make_async_copy(acc_spmem.at[pl.ds(base, WB)],
                                   rows_v.at[b, pl.ds(0, WB)], gsem.at[b])

    def _wdesc(q, b):
      base = s * ROWS_PER_TILE + q * WB
      return pltpu.make_async_copy(rows_v.at[b, pl.ds(0, WB)],
                                   agg_hbm.at[pl.ds(base, WB)], ssem.at[b])

    @pl.loop(0, WB_PER_TILE + 1)
    def _(q):
      @pl.when(q < WB_PER_TILE)
      def _():
        b = lax.rem(q, NBUF)

        @pl.when(q >= NBUF)
        def _():
          _wdesc(q - NBUF, b).wait()

        _rdesc(q, b).start()

      @pl.when(q >= 1)
      def _():
        b1 = lax.rem(q + (NBUF - 1), NBUF)
        _rdesc(q - 1, b1).wait()
        _wdesc(q - 1, b1).start()

    @pl.loop(WB_PER_TILE - NBUF, WB_PER_TILE)
    def _(q):
      _wdesc(q, lax.rem(q, NBUF)).wait()

  @pl.when(c == 0)
  def _():
    run(m_a_hbm, agg_a_hbm)

  @pl.when(c == 1)
  def _():
    run(m_b_hbm, agg_b_hbm)


def _full(shape):
  return pl.BlockSpec(shape, lambda i: tuple(0 for _ in shape))


def _rows(cols):
  return pl.BlockSpec((BLK, cols), lambda i: (i, 0))


def _tbl_body(emb_ref, wa_ref, wb_ref, ta_ref, tb_ref):
  e = emb_ref[...]
  ta_ref[...] = jnp.dot(e, wa_ref[...], preferred_element_type=jnp.float32)
  tb_ref[...] = jnp.dot(e, wb_ref[...], preferred_element_type=jnp.float32)


def _table_step(emb, wa, wb):
  return pl.pallas_call(
      _tbl_body,
      out_shape=(
          jax.ShapeDtypeStruct(emb.shape[:1] + (HALF,), jnp.float32),
          jax.ShapeDtypeStruct(emb.shape[:1] + (HALF,), jnp.float32),
      ),
  )(emb, wa, wb)


def _gh_body(h_ref, whh_ref, bhh_ref, gh_ref):
  gh_ref[...] = (jnp.dot(h_ref[...], whh_ref[...],
                         preferred_element_type=jnp.float32) + bhh_ref[...])


def _gh_step(h, whh_t, b_hh):
  return pl.pallas_call(
      _gh_body,
      grid=(N_BLOCKS,),
      in_specs=[_rows(HIDDEN), _full((HIDDEN, 3 * HIDDEN)),
                _full((1, 3 * HIDDEN))],
      out_specs=_rows(3 * HIDDEN),
      out_shape=jax.ShapeDtypeStruct((N_NODES, 3 * HIDDEN), jnp.float32),
  )(h, whh_t, b_hh)


def _gru_math(agg_a, agg_b, h, gh, wih_t_a, wih_t_b, b_ih):
  gi = (jnp.dot(agg_a, wih_t_a, preferred_element_type=jnp.float32)
        + jnp.dot(agg_b, wih_t_b, preferred_element_type=jnp.float32) + b_ih)
  r = jax.nn.sigmoid(gi[:, 0:HIDDEN] + gh[:, 0:HIDDEN])
  z = jax.nn.sigmoid(gi[:, HIDDEN:2 * HIDDEN] + gh[:, HIDDEN:2 * HIDDEN])
  n = jnp.tanh(gi[:, 2 * HIDDEN:] + r * gh[:, 2 * HIDDEN:])
  return (1.0 - z) * n + z * h


def _gru_next_body(aa_ref, ab_ref, h_ref, gh_ref, wiha_ref, wihb_ref,
                   bih_ref, wna_ref, wnb_ref,
                   h_out_ref, ma_ref, mb_ref):
  h_new = _gru_math(aa_ref[...], ab_ref[...], h_ref[...], gh_ref[...],
                    wiha_ref[...], wihb_ref[...], bih_ref[...])
  h_out_ref[...] = h_new
  ma_ref[...] = jnp.dot(h_new, wna_ref[...], preferred_element_type=jnp.float32)
  mb_ref[...] = jnp.dot(h_new, wnb_ref[...], preferred_element_type=jnp.float32)


def _gru_next(agg_a, agg_b, h, gh, wih_t_a, wih_t_b, b_ih, wna, wnb):
  return pl.pallas_call(
      _gru_next_body,
      grid=(N_BLOCKS,),
      in_specs=[
          _rows(HALF), _rows(HALF), _rows(HIDDEN), _rows(3 * HIDDEN),
          _full((HALF, 3 * HIDDEN)), _full((HALF, 3 * HIDDEN)),
          _full((1, 3 * HIDDEN)),
          _full((HIDDEN, HALF)), _full((HIDDEN, HALF)),
      ],
      out_specs=(_rows(HIDDEN), _rows(HALF), _rows(HALF)),
      out_shape=(
          jax.ShapeDtypeStruct((N_NODES, HIDDEN), jnp.float32),
          jax.ShapeDtypeStruct((N_NODES, HALF), jnp.float32),
          jax.ShapeDtypeStruct((N_NODES, HALF), jnp.float32),
      ),
  )(agg_a, agg_b, h, gh, wih_t_a, wih_t_b, b_ih, wna, wnb)


def _final_body(aa_ref, ab_ref, h_ref, gh_ref, batch_ref,
                wiha_ref, wihb_ref, bih_ref,
                l1w_ref, l1b_ref, ow_ref, ob_ref,
                out_ref, sums_acc, cnt_acc):
  i = pl.program_id(0)

  @pl.when(i == 0)
  def _():
    sums_acc[...] = jnp.zeros_like(sums_acc)
    cnt_acc[...] = jnp.zeros_like(cnt_acc)

  h_new = _gru_math(aa_ref[...], ab_ref[...], h_ref[...], gh_ref[...],
                    wiha_ref[...], wihb_ref[...], bih_ref[...])
  hr = jnp.maximum(h_new, 0.0)
  b = batch_ref[0, 0, :]
  onehot = (b[:, None]
            == lax.broadcasted_iota(jnp.int32, (BLK, NUM_GRAPHS), 1)
            ).astype(jnp.float32)
  sums_acc[...] += jax.lax.dot_general(
      hr, onehot, (((0,), (0,)), ((), ())),
      preferred_element_type=jnp.float32)
  cnt_acc[...] += jnp.sum(onehot, axis=0, keepdims=True)

  @pl.when(i == N_BLOCKS - 1)
  def _():
    recip = 1.0 / jnp.maximum(cnt_acc[...], 1.0)
    pooled_t = sums_acc[...] * recip
    y1_t = jnp.maximum(
        jnp.dot(l1w_ref[...], pooled_t, preferred_element_type=jnp.float32)
        + l1b_ref[...], 0.0)
    out_ref[...] = (jnp.dot(ow_ref[...], y1_t,
                            preferred_element_type=jnp.float32)
                    + ob_ref[...])


def _final_step(agg_a, agg_b, h, gh, batch3d, wih_t_a, wih_t_b, b_ih,
                l1w, l1b_col, ow_pad, ob_col):
  return pl.pallas_call(
      _final_body,
      grid=(N_BLOCKS,),
      in_specs=[
          _rows(HALF), _rows(HALF), _rows(HIDDEN), _rows(3 * HIDDEN),
          pl.BlockSpec((1, 1, BLK), lambda i: (i, 0, 0)),
          _full((HALF, 3 * HIDDEN)), _full((HALF, 3 * HIDDEN)),
          _full((1, 3 * HIDDEN)),
          _full((HIDDEN, HIDDEN)), _full((HIDDEN, 1)),
          _full((8, HIDDEN)), _full((8, 1)),
      ],
      out_specs=pl.BlockSpec((8, NUM_GRAPHS), lambda i: (0, 0)),
      out_shape=jax.ShapeDtypeStruct((8, NUM_GRAPHS), jnp.float32),
      scratch_shapes=[
          pltpu.VMEM((HIDDEN, NUM_GRAPHS), jnp.float32),
          pltpu.VMEM((1, NUM_GRAPHS), jnp.float32),
      ],
  )(agg_a, agg_b, h, gh, batch3d, wih_t_a, wih_t_b, b_ih,
    l1w, l1b_col, ow_pad, ob_col)


def kernel(x, edge_index, batch, emb_table, ggnn_weight, W_ih, W_hh,
           b_ih, b_hh, lin1_W, lin1_b, out_W, out_b):
  x = jnp.asarray(x, jnp.int32)
  edge_index = jnp.asarray(edge_index, jnp.int32)
  batch = jnp.asarray(batch, jnp.int32)

  x2d = x.reshape(X_CHUNKS, XC)
  src2d = edge_index[0].reshape(N_CHUNKS, EC)
  dst2d = edge_index[1].reshape(N_CHUNKS, EC)
  batch3d = batch.reshape(N_BLOCKS, 1, BLK)

  wih_t = W_ih.T
  wih_t_a = wih_t[:HALF]
  wih_t_b = wih_t[HALF:]
  whh_t = W_hh.T
  b_ih_r = b_ih.reshape(1, 3 * HIDDEN)
  b_hh_r = b_hh.reshape(1, 3 * HIDDEN)
  w_step = [(ggnn_weight[i][:, :HALF], ggnn_weight[i][:, HALF:])
            for i in range(STEPS)]
  l1b_col = lin1_b.reshape(HIDDEN, 1)
  ow_pad = jnp.zeros((8, HIDDEN), jnp.float32).at[:out_W.shape[0]].set(out_W)
  ob_col = jnp.zeros((8, 1), jnp.float32).at[:out_b.shape[0], 0].set(out_b)

  t0a, t0b = _table_step(emb_table, *w_step[0])
  h, m_a, m_b = _embed_kernel(x2d, emb_table, t0a, t0b)

  for i in range(STEPS - 1):
    agg_a, agg_b = _scatter_kernel(m_a, m_b, src2d, dst2d)
    gh = _gh_step(h, whh_t, b_hh_r)
    h, m_a, m_b = _gru_next(agg_a, agg_b, h, gh, wih_t_a, wih_t_b,
                            b_ih_r, *w_step[i + 1])
  agg_a, agg_b = _scatter_kernel(m_a, m_b, src2d, dst2d)
  gh = _gh_step(h, whh_t, b_hh_r)
  out_pad = _final_step(agg_a, agg_b, h, gh, batch3d, wih_t_a, wih_t_b,
                        b_ih_r, lin1_W, l1b_col, ow_pad, ob_col)
  return out_pad[:out_W.shape[0], :].T


# speedup vs baseline: 1.1094x; 1.1094x over previous
import functools

import jax
import jax.numpy as jnp
from jax import lax
from jax.experimental import pallas as pl
from jax.experimental.pallas import tpu as pltpu
from jax.experimental.pallas import tpu_sc as plsc

N_NODES = 50000
N_EDGES = 800000
HIDDEN = 64
HALF = 32
NUM_GRAPHS = 128
STEPS = 3

NUM_CORES = 2
NUM_SUBCORES = 16

EC = 250
N_CHUNKS = N_EDGES // EC
CHUNKS_PER_TILE = N_CHUNKS // NUM_SUBCORES
IBLK = 4
N_IBLKS = CHUNKS_PER_TILE // IBLK
NBUF = 3

ROWS_PER_TILE = N_NODES // NUM_SUBCORES
WB = 125
WB_PER_TILE = ROWS_PER_TILE // WB

XC = 200
X_CHUNKS = N_NODES // XC
X_ITERS = -(-X_CHUNKS // (NUM_CORES * NUM_SUBCORES))

BLK = 5000
N_BLOCKS = N_NODES // BLK

_sc_mesh = plsc.VectorSubcoreMesh(
    core_axis_name="c", subcore_axis_name="s",
    num_cores=NUM_CORES, num_subcores=NUM_SUBCORES)

_sc_params = pltpu.CompilerParams(use_tc_tiling_on_sc=False)


@functools.partial(
    pl.kernel,
    out_type=(
        jax.ShapeDtypeStruct((N_NODES, HIDDEN), jnp.float32),
        jax.ShapeDtypeStruct((N_NODES, HALF), jnp.float32),
        jax.ShapeDtypeStruct((N_NODES, HALF), jnp.float32),
    ),
    mesh=_sc_mesh,
    compiler_params=_sc_params,
    scratch_types=[
        pltpu.VMEM((XC,), jnp.int32),
        pltpu.VMEM((XC, HIDDEN), jnp.float32),
        pltpu.VMEM((XC, HALF), jnp.float32),
        pltpu.VMEM((XC, HALF), jnp.float32),
        pltpu.SemaphoreType.DMA,
        pltpu.SemaphoreType.DMA,
        pltpu.SemaphoreType.DMA,
    ],
)
def _embed_kernel(x2d_hbm, table_hbm, t0a_hbm, t0b_hbm,
                  h_hbm, ma_hbm, mb_hbm,
                  idx_v, rows_v, ra_v, rb_v, sem, sema, semb):
  c = lax.axis_index("c")
  s = lax.axis_index("s")
  w = s * NUM_CORES + c

  @pl.loop(0, X_ITERS)
  def _(k):
    ci = k * (NUM_CORES * NUM_SUBCORES) + w

    @pl.when(ci < X_CHUNKS)
    def _():
      pltpu.sync_copy(x2d_hbm.at[ci], idx_v)
      ga = pltpu.async_copy(t0a_hbm.at[idx_v], ra_v, sema)
      gb = pltpu.async_copy(t0b_hbm.at[idx_v], rb_v, semb)
      gh = pltpu.async_copy(table_hbm.at[idx_v], rows_v, sem)
      ga.wait()
      pltpu.async_copy(ra_v, ma_hbm.at[pl.ds(ci * XC, XC)], sema).wait()
      gb.wait()
      pltpu.async_copy(rb_v, mb_hbm.at[pl.ds(ci * XC, XC)], semb).wait()
      gh.wait()
      pltpu.async_copy(rows_v, h_hbm.at[pl.ds(ci * XC, XC)], sem).wait()


@functools.partial(
    pl.kernel,
    out_type=(
        jax.ShapeDtypeStruct((N_NODES, HALF), jnp.float32),
        jax.ShapeDtypeStruct((N_NODES, HALF), jnp.float32),
    ),
    mesh=_sc_mesh,
    compiler_params=_sc_params,
    scratch_types=[
        pltpu.VMEM_SHARED((N_NODES, HALF), jnp.float32),
        pltpu.VMEM((NBUF, IBLK, EC), jnp.int32),
        pltpu.VMEM((NBUF, IBLK, EC), jnp.int32),
        pltpu.VMEM((NBUF, EC, HALF), jnp.float32),
        pltpu.SemaphoreType.DMA((NBUF,)),
        pltpu.SemaphoreType.DMA((NBUF,)),
        pltpu.SemaphoreType.DMA((NBUF,)),
    ],
)
def _scatter_kernel(m_a_hbm, m_b_hbm, src_hbm, dst_hbm,
                    agg_a_hbm, agg_b_hbm,
                    acc_spmem, sidx_v, didx_v, rows_v, isem, gsem, ssem):
  c = lax.axis_index("c")
  s = lax.axis_index("s")

  z16 = jnp.zeros((16,), jnp.float32)

  @pl.loop(0, EC)
  def _(i):
    rows_v[0, i, pl.ds(0, 16)] = z16
    rows_v[0, i, pl.ds(16, 16)] = z16

  zrow = rows_v.at[0, pl.ds(0, WB)]

  def _zdesc(q):
    base = s * ROWS_PER_TILE + q * WB
    return pltpu.make_async_copy(zrow, acc_spmem.at[pl.ds(base, WB)],
                                 gsem.at[0])

  @pl.loop(0, WB_PER_TILE)
  def _(q):
    _zdesc(q).start()

  @pl.loop(0, WB_PER_TILE)
  def _(q):
    _zdesc(q).wait()

  plsc.subcore_barrier()

  def run(m_hbm, agg_hbm):
    chunk0 = s * CHUNKS_PER_TILE

    def _idesc(blk, ib):
      c0 = chunk0 + blk * IBLK
      return (
          pltpu.make_async_copy(src_hbm.at[pl.ds(c0, IBLK)], sidx_v.at[ib],
                                isem.at[ib]),
          pltpu.make_async_copy(dst_hbm.at[pl.ds(c0, IBLK)], didx_v.at[ib],
                                isem.at[ib]),
      )

    def _gdesc(b, ib, j):
      return pltpu.make_async_copy(m_hbm.at[sidx_v.at[ib, j]], rows_v.at[b],
                                   gsem.at[b])

    def _sdesc(b, ib, j):
      return pltpu.make_async_copy(rows_v.at[b], acc_spmem.at[didx_v.at[ib, j]],
                                   ssem.at[b])

    for d in _idesc(0, 0):
      d.start()

    @pl.loop(0, N_IBLKS)
    def _(blk):
      ib = lax.rem(blk, NBUF)
      ib_prev = lax.rem(blk + (NBUF - 1), NBUF)
      for d in _idesc(blk, ib):
        d.wait()

      @pl.when(blk + 1 < N_IBLKS)
      def _():
        for d in _idesc(blk + 1, lax.rem(blk + 1, NBUF)):
          d.start()

      for j in range(IBLK):
        k = blk * IBLK + j
        b = lax.rem(k, NBUF)

        if j >= NBUF:
          _sdesc(b, ib, j - NBUF).wait()
        else:
          @pl.when(blk >= 1)
          def _():
            _sdesc(b, ib_prev, j + IBLK - NBUF).wait()

        _gdesc(b, ib, j).start()

        if j >= 2:
          b2 = lax.rem(k - 2, NBUF)
          _gdesc(b2, ib, j - 2).wait()
          _sdesc(b2, ib, j - 2).start(add=True)
        else:
          @pl.when(blk >= 1)
          def _():
            b2 = lax.rem(k + (NBUF - 2), NBUF)
            _gdesc(b2, ib_prev, j + IBLK - 2).wait()
            _sdesc(b2, ib_prev, j + IBLK - 2).start(add=True)

    ib_last = (N_IBLKS - 1) % NBUF
    for k in (CHUNKS_PER_TILE - 2, CHUNKS_PER_TILE - 1):
      b = k % NBUF
      j = k - (N_IBLKS - 1) * IBLK
      _gdesc(b, ib_last, j).wait()
      _sdesc(b, ib_last, j).start(add=True)
    for k in range(CHUNKS_PER_TILE - NBUF, CHUNKS_PER_TILE):
      b = k % NBUF
      j = k - (N_IBLKS - 1) * IBLK
      _sdesc(b, ib_last, max(j, 0)).wait()

    plsc.subcore_barrier()

    def _rdesc(q, b):
      base = s * ROWS_PER_TILE + q * WB
      return pltpu.make_async_copy(acc_spmem.at[pl.ds(base, WB)],
                                   rows_v.at[b, pl.ds(0, WB)], gsem.at[b])

    def _wdesc(q, b):
      base = s * ROWS_PER_TILE + q * WB
      return pltpu.make_async_copy(rows_v.at[b, pl.ds(0, WB)],
                                   agg_hbm.at[pl.ds(base, WB)], ssem.at[b])

    @pl.loop(0, WB_PER_TILE + 1)
    def _(q):
      @pl.when(q < WB_PER_TILE)
      def _():
        b = lax.rem(q, NBUF)

        @pl.when(q >= NBUF)
        def _():
          _wdesc(q - NBUF, b).wait()

        _rdesc(q, b).start()

      @pl.when(q >= 1)
      def _():
        b1 = lax.rem(q + (NBUF - 1), NBUF)
        _rdesc(q - 1, b1).wait()
        _wdesc(q - 1, b1).start()

    @pl.loop(WB_PER_TILE - NBUF, WB_PER_TILE)
    def _(q):
      _wdesc(q, lax.rem(q, NBUF)).wait()

  @pl.when(c == 0)
  def _():
    run(m_a_hbm, agg_a_hbm)

  @pl.when(c == 1)
  def _():
    run(m_b_hbm, agg_b_hbm)


def _full(shape):
  return pl.BlockSpec(shape, lambda i: tuple(0 for _ in shape))


def _rows(cols):
  return pl.BlockSpec((BLK, cols), lambda i: (i, 0))


def _tbl_body(emb_ref, wa_ref, wb_ref, ta_ref, tb_ref):
  e = emb_ref[...]
  ta_ref[...] = jnp.dot(e, wa_ref[...], preferred_element_type=jnp.float32)
  tb_ref[...] = jnp.dot(e, wb_ref[...], preferred_element_type=jnp.float32)


def _table_step(emb, wa, wb):
  return pl.pallas_call(
      _tbl_body,
      out_shape=(
          jax.ShapeDtypeStruct(emb.shape[:1] + (HALF,), jnp.float32),
          jax.ShapeDtypeStruct(emb.shape[:1] + (HALF,), jnp.float32),
      ),
  )(emb, wa, wb)


def _gru_math(agg_a, agg_b, h, wih_t_a, wih_t_b, whh_t, b_ih, b_hh):
  gi = (jnp.dot(agg_a, wih_t_a, preferred_element_type=jnp.float32)
        + jnp.dot(agg_b, wih_t_b, preferred_element_type=jnp.float32) + b_ih)
  gh = jnp.dot(h, whh_t, preferred_element_type=jnp.float32) + b_hh
  r = jax.nn.sigmoid(gi[:, 0:HIDDEN] + gh[:, 0:HIDDEN])
  z = jax.nn.sigmoid(gi[:, HIDDEN:2 * HIDDEN] + gh[:, HIDDEN:2 * HIDDEN])
  n = jnp.tanh(gi[:, 2 * HIDDEN:] + r * gh[:, 2 * HIDDEN:])
  return (1.0 - z) * n + z * h


def _gru_next_body(aa_ref, ab_ref, h_ref, wiha_ref, wihb_ref, whh_ref,
                   bih_ref, bhh_ref, wna_ref, wnb_ref,
                   h_out_ref, ma_ref, mb_ref):
  h_new = _gru_math(aa_ref[...], ab_ref[...], h_ref[...],
                    wiha_ref[...], wihb_ref[...], whh_ref[...],
                    bih_ref[...], bhh_ref[...])
  h_out_ref[...] = h_new
  ma_ref[...] = jnp.dot(h_new, wna_ref[...], preferred_element_type=jnp.float32)
  mb_ref[...] = jnp.dot(h_new, wnb_ref[...], preferred_element_type=jnp.float32)


def _gru_next(agg_a, agg_b, h, wih_t_a, wih_t_b, whh_t, b_ih, b_hh, wna, wnb):
  return pl.pallas_call(
      _gru_next_body,
      grid=(N_BLOCKS,),
      in_specs=[
          _rows(HALF), _rows(HALF), _rows(HIDDEN),
          _full((HALF, 3 * HIDDEN)), _full((HALF, 3 * HIDDEN)),
          _full((HIDDEN, 3 * HIDDEN)),
          _full((1, 3 * HIDDEN)), _full((1, 3 * HIDDEN)),
          _full((HIDDEN, HALF)), _full((HIDDEN, HALF)),
      ],
      out_specs=(_rows(HIDDEN), _rows(HALF), _rows(HALF)),
      out_shape=(
          jax.ShapeDtypeStruct((N_NODES, HIDDEN), jnp.float32),
          jax.ShapeDtypeStruct((N_NODES, HALF), jnp.float32),
          jax.ShapeDtypeStruct((N_NODES, HALF), jnp.float32),
      ),
  )(agg_a, agg_b, h, wih_t_a, wih_t_b, whh_t, b_ih, b_hh, wna, wnb)


def _final_body(aa_ref, ab_ref, h_ref, batch_ref,
                wiha_ref, wihb_ref, whh_ref, bih_ref, bhh_ref,
                l1w_ref, l1b_ref, ow_ref, ob_ref,
                out_ref, sums_acc, cnt_acc):
  i = pl.program_id(0)

  @pl.when(i == 0)
  def _():
    sums_acc[...] = jnp.zeros_like(sums_acc)
    cnt_acc[...] = jnp.zeros_like(cnt_acc)

  h_new = _gru_math(aa_ref[...], ab_ref[...], h_ref[...],
                    wiha_ref[...], wihb_ref[...], whh_ref[...],
                    bih_ref[...], bhh_ref[...])
  hr = jnp.maximum(h_new, 0.0)
  b = batch_ref[0, 0, :]
  onehot = (b[:, None]
            == lax.broadcasted_iota(jnp.int32, (BLK, NUM_GRAPHS), 1)
            ).astype(jnp.float32)
  sums_acc[...] += jax.lax.dot_general(
      hr, onehot, (((0,), (0,)), ((), ())),
      preferred_element_type=jnp.float32)
  cnt_acc[...] += jnp.sum(onehot, axis=0, keepdims=True)

  @pl.when(i == N_BLOCKS - 1)
  def _():
    recip = 1.0 / jnp.maximum(cnt_acc[...], 1.0)
    pooled_t = sums_acc[...] * recip
    y1_t = jnp.maximum(
        jnp.dot(l1w_ref[...], pooled_t, preferred_element_type=jnp.float32)
        + l1b_ref[...], 0.0)
    out_ref[...] = (jnp.dot(ow_ref[...], y1_t,
                            preferred_element_type=jnp.float32)
                    + ob_ref[...])


def _final_step(agg_a, agg_b, h, batch3d, wih_t_a, wih_t_b, whh_t, b_ih, b_hh,
                l1w, l1b_col, ow_pad, ob_col):
  return pl.pallas_call(
      _final_body,
      grid=(N_BLOCKS,),
      in_specs=[
          _rows(HALF), _rows(HALF), _rows(HIDDEN),
          pl.BlockSpec((1, 1, BLK), lambda i: (i, 0, 0)),
          _full((HALF, 3 * HIDDEN)), _full((HALF, 3 * HIDDEN)),
          _full((HIDDEN, 3 * HIDDEN)),
          _full((1, 3 * HIDDEN)), _full((1, 3 * HIDDEN)),
          _full((HIDDEN, HIDDEN)), _full((HIDDEN, 1)),
          _full((8, HIDDEN)), _full((8, 1)),
      ],
      out_specs=pl.BlockSpec((8, NUM_GRAPHS), lambda i: (0, 0)),
      out_shape=jax.ShapeDtypeStruct((8, NUM_GRAPHS), jnp.float32),
      scratch_shapes=[
          pltpu.VMEM((HIDDEN, NUM_GRAPHS), jnp.float32),
          pltpu.VMEM((1, NUM_GRAPHS), jnp.float32),
      ],
  )(agg_a, agg_b, h, batch3d, wih_t_a, wih_t_b, whh_t, b_ih, b_hh,
    l1w, l1b_col, ow_pad, ob_col)


def kernel(x, edge_index, batch, emb_table, ggnn_weight, W_ih, W_hh,
           b_ih, b_hh, lin1_W, lin1_b, out_W, out_b):
  x = jnp.asarray(x, jnp.int32)
  edge_index = jnp.asarray(edge_index, jnp.int32)
  batch = jnp.asarray(batch, jnp.int32)

  x2d = x.reshape(X_CHUNKS, XC)
  src2d = edge_index[0].reshape(N_CHUNKS, EC)
  dst2d = edge_index[1].reshape(N_CHUNKS, EC)
  batch3d = batch.reshape(N_BLOCKS, 1, BLK)

  wih_t = W_ih.T
  wih_t_a = wih_t[:HALF]
  wih_t_b = wih_t[HALF:]
  whh_t = W_hh.T
  b_ih_r = b_ih.reshape(1, 3 * HIDDEN)
  b_hh_r = b_hh.reshape(1, 3 * HIDDEN)
  w_step = [(ggnn_weight[i][:, :HALF], ggnn_weight[i][:, HALF:])
            for i in range(STEPS)]
  l1b_col = lin1_b.reshape(HIDDEN, 1)
  ow_pad = jnp.zeros((8, HIDDEN), jnp.float32).at[:out_W.shape[0]].set(out_W)
  ob_col = jnp.zeros((8, 1), jnp.float32).at[:out_b.shape[0], 0].set(out_b)

  t0a, t0b = _table_step(emb_table, *w_step[0])
  h, m_a, m_b = _embed_kernel(x2d, emb_table, t0a, t0b)

  for i in range(STEPS - 1):
    agg_a, agg_b = _scatter_kernel(m_a, m_b, src2d, dst2d)
    h, m_a, m_b = _gru_next(agg_a, agg_b, h, wih_t_a, wih_t_b, whh_t,
                            b_ih_r, b_hh_r, *w_step[i + 1])
  agg_a, agg_b = _scatter_kernel(m_a, m_b, src2d, dst2d)
  out_pad = _final_step(agg_a, agg_b, h, batch3d, wih_t_a, wih_t_b, whh_t,
                        b_ih_r, b_hh_r, lin1_W, l1b_col, ow_pad, ob_col)
  return out_pad[:out_W.shape[0], :].T
